# Initial kernel scaffold; baseline (speedup 1.0000x reference)
import jax
import jax.numpy as jnp
from jax.experimental import pallas as pl


def kernel(q, fmap, ref_xy, W_v, W_off, b_off, W_w, b_w, W_out, b_out):
    raise NotImplementedError("write your pallas kernel here")


# speedup vs baseline: 38.2302x; 38.2302x over previous
import jax
import jax.numpy as jnp
from jax.experimental import pallas as pl

_H = 8
_P = 16
_R = 0.08
_TT = 256
_G = 576
_DH = 32


def _body(q_ref, fm_ref, rxy_ref, wv_ref, woff_ref, boff_ref, ww_ref, bw_ref,
          wout_ref, bout_ref, out_ref):
    h = pl.program_id(2)
    q = q_ref[0]
    fm = fm_ref[0]
    wv = wv_ref[0]
    V = jax.lax.dot_general(wv, fm, (((1,), (0,)), ((), ())),
                            preferred_element_type=jnp.float32)
    off = jax.lax.dot_general(q, woff_ref[0], (((1,), (1,)), ((), ())),
                              preferred_element_type=jnp.float32) + boff_ref[0]
    wl = jax.lax.dot_general(q, ww_ref[0], (((1,), (1,)), ((), ())),
                             preferred_element_type=jnp.float32) + bw_ref[0]
    wl = wl - jnp.max(wl, axis=1, keepdims=True)
    we = jnp.exp(wl)
    w = we / jnp.sum(we, axis=1, keepdims=True)

    rxy = rxy_ref[0]
    rx = rxy[:, 0:1]
    ry = rxy[:, 1:2]
    s_i = jax.lax.broadcasted_iota(jnp.int32, (1, _G), 1)
    ys = (s_i // 24).astype(jnp.float32)
    xs = (s_i % 24).astype(jnp.float32)

    acc = jnp.zeros((_TT, _G), jnp.float32)
    for p in range(_P):
        xf = (rx + _R * off[:, 2 * p:2 * p + 1]) * 23.0
        yf = (ry + _R * off[:, 2 * p + 1:2 * p + 2]) * 23.0
        tx = jnp.maximum(1.0 - jnp.abs(xs - xf), 0.0)
        ty = jnp.maximum(1.0 - jnp.abs(ys - yf), 0.0)
        acc = acc + (w[:, p:p + 1] * tx) * ty

    ctx = jax.lax.dot_general(acc, V, (((1,), (1,)), ((), ())),
                              preferred_element_type=jnp.float32)
    contrib = jax.lax.dot_general(ctx, wout_ref[0], (((1,), (0,)), ((), ())),
                                  preferred_element_type=jnp.float32)

    @pl.when(h == 0)
    def _():
        out_ref[0] = contrib + bout_ref[...]

    @pl.when(h != 0)
    def _():
        out_ref[0] += contrib


def kernel(q, fmap, ref_xy, W_v, W_off, b_off, W_w, b_w, W_out, b_out):
    B, T, D = q.shape
    C = fmap.shape[1]
    fm = fmap.reshape(B, C, _G)
    wv = W_v.reshape(_H, _DH, C)
    woff = W_off.reshape(_H, 2 * _P, D)
    boff = b_off.reshape(_H, 1, 2 * _P)
    ww = W_w.reshape(_H, _P, D)
    bw = b_w.reshape(_H, 1, _P)
    wout = W_out.T.reshape(_H, _DH, D)
    bout = b_out.reshape(1, D)
    nt = T // _TT
    return pl.pallas_call(
        _body,
        grid=(B, nt, _H),
        in_specs=[
            pl.BlockSpec((1, _TT, D), lambda b, t, h: (b, t, 0)),
            pl.BlockSpec((1, C, _G), lambda b, t, h: (b, 0, 0)),
            pl.BlockSpec((1, _TT, 2), lambda b, t, h: (b, t, 0)),
            pl.BlockSpec((1, _DH, C), lambda b, t, h: (h, 0, 0)),
            pl.BlockSpec((1, 2 * _P, D), lambda b, t, h: (h, 0, 0)),
            pl.BlockSpec((1, 1, 2 * _P), lambda b, t, h: (h, 0, 0)),
            pl.BlockSpec((1, _P, D), lambda b, t, h: (h, 0, 0)),
            pl.BlockSpec((1, 1, _P), lambda b, t, h: (h, 0, 0)),
            pl.BlockSpec((1, _DH, D), lambda b, t, h: (h, 0, 0)),
            pl.BlockSpec((1, D), lambda b, t, h: (0, 0)),
        ],
        out_specs=pl.BlockSpec((1, _TT, D), lambda b, t, h: (b, t, 0)),
        out_shape=jax.ShapeDtypeStruct((B, T, D), jnp.float32),
    )(q, fm, ref_xy, wv, woff, boff, ww, bw, wout, bout)


# speedup vs baseline: 134.2812x; 3.5124x over previous
import jax
import jax.numpy as jnp
from jax.experimental import pallas as pl
from jax.experimental.pallas import tpu as pltpu

_H = 8
_P = 16
_R = 0.08
_TT = 256
_G = 576
_DH = 32


def _dot(a, b, dims):
    return jax.lax.dot_general(a, b, (dims, ((), ())),
                               preferred_element_type=jnp.float32)


def _body(q_ref, fm_ref, rxy_ref, wv_ref, woff_ref, boff_ref, ww_ref, bw_ref,
          wout_ref, bout_ref, out_ref, vs_ref):
    t_idx = pl.program_id(1)
    h = pl.program_id(2)

    @pl.when(t_idx == 0)
    def _():
        vs_ref[h] = _dot(wv_ref[0], fm_ref[0], ((1,), (0,)))

    q = q_ref[0]
    off = _dot(woff_ref[0], q, ((1,), (1,))) + boff_ref[0]
    wl = _dot(ww_ref[0], q, ((1,), (1,))) + bw_ref[0]
    wl = wl - jnp.max(wl, axis=0, keepdims=True)
    we = jnp.exp(wl)
    w = we / jnp.sum(we, axis=0, keepdims=True)

    rx = rxy_ref[0, 0:1, :]
    ry = rxy_ref[0, 1:2, :]
    xs = jax.lax.broadcasted_iota(jnp.int32, (24, 1), 0).astype(jnp.float32)

    acc = jnp.zeros((_G, _TT), jnp.float32)
    for p in range(_P):
        xf = (rx + _R * off[2 * p:2 * p + 1, :]) * 23.0
        yf = (ry + _R * off[2 * p + 1:2 * p + 2, :]) * 23.0
        tx = jnp.maximum(1.0 - jnp.abs(xs - xf), 0.0)
        ty = jnp.maximum(1.0 - jnp.abs(xs - yf), 0.0)
        wtx = w[p:p + 1, :] * tx
        txe = jnp.broadcast_to(wtx[None, :, :], (24, 24, _TT)).reshape(_G, _TT)
        tye = jnp.broadcast_to(ty[:, None, :], (24, 24, _TT)).reshape(_G, _TT)
        acc = acc + txe * tye

    ctx = _dot(vs_ref[h], acc, ((1,), (0,)))
    contrib = _dot(ctx, wout_ref[0], ((0,), (0,)))

    @pl.when(h == 0)
    def _():
        out_ref[0] = contrib + bout_ref[...]

    @pl.when(h != 0)
    def _():
        out_ref[0] += contrib


def kernel(q, fmap, ref_xy, W_v, W_off, b_off, W_w, b_w, W_out, b_out):
    B, T, D = q.shape
    C = fmap.shape[1]
    fm = fmap.reshape(B, C, _G)
    rxy = ref_xy.transpose(0, 2, 1)
    wv = W_v.reshape(_H, _DH, C)
    woff = W_off.reshape(_H, 2 * _P, D)
    boff = b_off.reshape(_H, 2 * _P, 1)
    ww = W_w.reshape(_H, _P, D)
    bw = b_w.reshape(_H, _P, 1)
    wout = W_out.T.reshape(_H, _DH, D)
    bout = b_out.reshape(1, D)
    nt = T // _TT
    return pl.pallas_call(
        _body,
        grid=(B, nt, _H),
        in_specs=[
            pl.BlockSpec((1, _TT, D), lambda b, t, h: (b, t, 0)),
            pl.BlockSpec((1, C, _G), lambda b, t, h: (b, 0, 0)),
            pl.BlockSpec((1, 2, _TT), lambda b, t, h: (b, 0, t)),
            pl.BlockSpec((1, _DH, C), lambda b, t, h: (h, 0, 0)),
            pl.BlockSpec((1, 2 * _P, D), lambda b, t, h: (h, 0, 0)),
            pl.BlockSpec((1, 2 * _P, 1), lambda b, t, h: (h, 0, 0)),
            pl.BlockSpec((1, _P, D), lambda b, t, h: (h, 0, 0)),
            pl.BlockSpec((1, _P, 1), lambda b, t, h: (h, 0, 0)),
            pl.BlockSpec((1, _DH, D), lambda b, t, h: (h, 0, 0)),
            pl.BlockSpec((1, D), lambda b, t, h: (0, 0)),
        ],
        out_specs=pl.BlockSpec((1, _TT, D), lambda b, t, h: (b, t, 0)),
        out_shape=jax.ShapeDtypeStruct((B, T, D), jnp.float32),
        scratch_shapes=[pltpu.VMEM((_H, _DH, _G), jnp.float32)],
    )(q, fm, rxy, wv, woff, boff, ww, bw, wout, bout)


# speedup vs baseline: 197.5640x; 1.4713x over previous
import jax
import jax.numpy as jnp
from jax.experimental import pallas as pl
from jax.experimental.pallas import tpu as pltpu

_H = 8
_P = 16
_R = 0.08
_TT = 2048
_G = 576
_DH = 32


def _dot(a, b, dims):
    return jax.lax.dot_general(a, b, (dims, ((), ())),
                               preferred_element_type=jnp.float32)


def _body(q_ref, fm_ref, rxy_ref, wv_ref, woff_ref, boff_ref, ww_ref, bw_ref,
          wout_ref, bout_ref, out_ref, vs_ref):
    t_idx = pl.program_id(1)
    h = pl.program_id(2)

    @pl.when(t_idx == 0)
    def _():
        vs_ref[h] = _dot(wv_ref[0], fm_ref[0], ((1,), (0,)))

    q = q_ref[0]
    off = _dot(woff_ref[0], q, ((1,), (1,))) + boff_ref[0]
    wl = _dot(ww_ref[0], q, ((1,), (1,))) + bw_ref[0]
    wl = wl - jnp.max(wl, axis=0, keepdims=True)
    we = jnp.exp(wl)
    w = we / jnp.sum(we, axis=0, keepdims=True)

    rx = rxy_ref[0, 0:1, :]
    ry = rxy_ref[0, 1:2, :]
    xs = jax.lax.broadcasted_iota(jnp.int32, (24, 1), 0).astype(jnp.float32)

    acc = jnp.zeros((24, 24, _TT), jnp.float32)
    for p in range(_P):
        xf = (rx + _R * off[2 * p:2 * p + 1, :]) * 23.0
        yf = (ry + _R * off[2 * p + 1:2 * p + 2, :]) * 23.0
        tx = jnp.maximum(1.0 - jnp.abs(xs - xf), 0.0)
        ty = jnp.maximum(1.0 - jnp.abs(xs - yf), 0.0)
        wtx = w[p:p + 1, :] * tx
        acc = acc + wtx[None, :, :] * ty[:, None, :]
    acc = acc.reshape(_G, _TT)

    ctx = _dot(vs_ref[h], acc, ((1,), (0,)))
    contrib = _dot(ctx, wout_ref[0], ((0,), (0,)))

    @pl.when(h == 0)
    def _():
        out_ref[0] = contrib + bout_ref[...]

    @pl.when(h != 0)
    def _():
        out_ref[0] += contrib


def kernel(q, fmap, ref_xy, W_v, W_off, b_off, W_w, b_w, W_out, b_out):
    B, T, D = q.shape
    C = fmap.shape[1]
    fm = fmap.reshape(B, C, _G)
    rxy = ref_xy.transpose(0, 2, 1)
    wv = W_v.reshape(_H, _DH, C)
    woff = W_off.reshape(_H, 2 * _P, D)
    boff = b_off.reshape(_H, 2 * _P, 1)
    ww = W_w.reshape(_H, _P, D)
    bw = b_w.reshape(_H, _P, 1)
    wout = W_out.T.reshape(_H, _DH, D)
    bout = b_out.reshape(1, D)
    nt = T // _TT
    return pl.pallas_call(
        _body,
        grid=(B, nt, _H),
        in_specs=[
            pl.BlockSpec((1, _TT, D), lambda b, t, h: (b, t, 0)),
            pl.BlockSpec((1, C, _G), lambda b, t, h: (b, 0, 0)),
            pl.BlockSpec((1, 2, _TT), lambda b, t, h: (b, 0, t)),
            pl.BlockSpec((1, _DH, C), lambda b, t, h: (h, 0, 0)),
            pl.BlockSpec((1, 2 * _P, D), lambda b, t, h: (h, 0, 0)),
            pl.BlockSpec((1, 2 * _P, 1), lambda b, t, h: (h, 0, 0)),
            pl.BlockSpec((1, _P, D), lambda b, t, h: (h, 0, 0)),
            pl.BlockSpec((1, _P, 1), lambda b, t, h: (h, 0, 0)),
            pl.BlockSpec((1, _DH, D), lambda b, t, h: (h, 0, 0)),
            pl.BlockSpec((1, D), lambda b, t, h: (0, 0)),
        ],
        out_specs=pl.BlockSpec((1, _TT, D), lambda b, t, h: (b, t, 0)),
        out_shape=jax.ShapeDtypeStruct((B, T, D), jnp.float32),
        scratch_shapes=[pltpu.VMEM((_H, _DH, _G), jnp.float32)],
    )(q, fm, rxy, wv, woff, boff, ww, bw, wout, bout)
